# baseline (speedup 1.0000x reference)
import functools

import jax
import jax.numpy as jnp
from jax import lax
from jax.experimental import pallas as pl
from jax.experimental.pallas import tpu as pltpu
from jax.experimental.pallas import tpu_sc as plsc

BATCH = 16384
HIST = 50
HISTP = 56
DIM = 32
NUM_CORES = 2
NUM_SUBCORES = 16
NUM_WORKERS = NUM_CORES * NUM_SUBCORES
B_PER_WORKER = BATCH // NUM_WORKERS
GB = 8
NUM_GROUPS = B_PER_WORKER // GB
NBUF = 4


@functools.partial(
    pl.kernel,
    mesh=plsc.VectorSubcoreMesh(core_axis_name="c", subcore_axis_name="s"),
    out_type=jax.ShapeDtypeStruct((BATCH, HIST, DIM), jnp.float32),
    scratch_types=[
        pltpu.VMEM((B_PER_WORKER, HISTP), jnp.int32),
        pltpu.VMEM((NBUF, GB, HISTP, DIM), jnp.float32),
    ]
    + [pltpu.SemaphoreType.DMA] * (2 * NBUF),
    compiler_params=pltpu.CompilerParams(use_tc_tiling_on_sc=False),
)
def _gather_kernel(table_hbm, idx_hbm, out_hbm, idx_v, rows_v, *sems):
    gsem = sems[:NBUF]
    osem = sems[NBUF:]
    wid = lax.axis_index("s") * NUM_CORES + lax.axis_index("c")
    bbase = wid * B_PER_WORKER
    pltpu.sync_copy(idx_hbm.at[pl.ds(bbase, B_PER_WORKER)], idx_v)

    def fire(g, b):
        for j in range(GB):
            pltpu.async_copy(
                table_hbm.at[idx_v.at[g * GB + j]],
                rows_v.at[b, j],
                gsem[b],
            )

    def drain_gathers(g, b):
        for j in range(GB):
            pltpu.make_async_copy(
                table_hbm.at[idx_v.at[g * GB + j]],
                rows_v.at[b, j],
                gsem[b],
            ).wait()

    def fire_out(s, b):
        for j in range(GB):
            pltpu.async_copy(
                rows_v.at[b, j, pl.ds(0, HIST)],
                out_hbm.at[bbase + s * GB + j],
                osem[b],
            )

    def drain_out(s, b):
        for j in range(GB):
            pltpu.make_async_copy(
                rows_v.at[b, j, pl.ds(0, HIST)],
                out_hbm.at[bbase + s * GB + j],
                osem[b],
            ).wait()

    for g in range(NBUF - 1):
        fire(g, g)

    def step(s, b):
        drain_gathers(s, b)
        fire_out(s, b)
        bn = (b + NBUF - 1) % NBUF

        @pl.when(s > 0)
        def _():
            drain_out(s - 1, bn)

        @pl.when(s + NBUF - 1 < NUM_GROUPS)
        def _():
            fire(s + NBUF - 1, bn)

    def body(p, carry):
        for b in range(NBUF):
            step(p * NBUF + b, b)
        return carry

    lax.fori_loop(0, NUM_GROUPS // NBUF, body, 0)
    drain_out(NUM_GROUPS - 1, (NUM_GROUPS - 1) % NBUF)


def kernel(input, weight):
    idx = input.astype(jnp.int32)
    idxp = jnp.concatenate([idx, idx[:, : HISTP - HIST]], axis=1)
    return _gather_kernel(weight, idxp)
